# Initial kernel scaffold; baseline (speedup 1.0000x reference)
import functools

import jax
import jax.numpy as jnp
from jax import lax
from jax.experimental import pallas as pl
from jax.experimental.pallas import tpu as pltpu
from jax.experimental.pallas import tpu_sc as plsc


def _dist_argmin_body(f_ref, e_ref, f2_ref, e2_ref, idx_ref, acc_ref):
    n = e_ref.shape[0]
    dot = lax.dot_general(2.0 * f_ref[...], e_ref[...],
                          (((1,), (1,)), ((), ())),
                          preferred_element_type=jnp.float32)
    dist = (f2_ref[...] - dot) + e2_ref[...]
    m = jnp.min(dist, axis=1, keepdims=True)
    iota = lax.broadcasted_iota(jnp.int32, dist.shape, 1)
    idx_ref[...] = jnp.min(jnp.where(dist == m, iota, n), axis=1, keepdims=True)

    @pl.when(pl.program_id(0) == 0)
    def _():
        acc_ref[0, 0] = 0.0

    acc_ref[0, 0] += jnp.sum(m)


def _dist_argmin(flat, embedding, f2, e2, bm):
    m, d = flat.shape
    n = embedding.shape[0]
    return pl.pallas_call(
        _dist_argmin_body,
        grid=(m // bm,),
        in_specs=[
            pl.BlockSpec((bm, d), lambda i: (i, 0)),
            pl.BlockSpec((n, d), lambda i: (0, 0)),
            pl.BlockSpec((bm, 1), lambda i: (i, 0)),
            pl.BlockSpec((1, n), lambda i: (0, 0)),
        ],
        out_specs=[
            pl.BlockSpec((bm, 1), lambda i: (i, 0)),
            pl.BlockSpec((1, 1), lambda i: (0, 0), memory_space=pltpu.SMEM),
        ],
        out_shape=[
            jax.ShapeDtypeStruct((m, 1), jnp.int32),
            jax.ShapeDtypeStruct((1, 1), jnp.float32),
        ],
    )(flat, embedding, f2, e2)


def _make_sc_gather(v, d, b):
    info = plsc.get_sparse_core_info()
    nc, ns = info.num_cores, info.num_subcores
    nw = nc * ns
    assert b % (8 * nw) == 0
    b_per_w = b // nw
    chunks = 1
    while (b_per_w // chunks) * d * 4 > 300_000 or b_per_w % chunks:
        chunks += 1
    rows_per_chunk = b_per_w // chunks
    mesh = plsc.VectorSubcoreMesh(core_axis_name="c", subcore_axis_name="s")

    @functools.partial(
        pl.kernel,
        out_type=jax.ShapeDtypeStruct((b, d), jnp.float32),
        mesh=mesh,
        scratch_types=[
            pltpu.VMEM((chunks, rows_per_chunk), jnp.int32),
            pltpu.VMEM((rows_per_chunk, d), jnp.float32),
            pltpu.SemaphoreType.DMA,
        ],
    )
    def gather(table_hbm, idx_hbm, out_hbm, idx_v, rows_v, sem):
        wid = lax.axis_index("s") * nc + lax.axis_index("c")
        base = wid * b_per_w
        for c in range(chunks):
            off = base + c * rows_per_chunk
            pltpu.sync_copy(idx_hbm.at[pl.ds(off, rows_per_chunk)], idx_v.at[c])
            pltpu.async_copy(table_hbm.at[idx_v.at[c]], rows_v, sem).wait()
            pltpu.sync_copy(rows_v, out_hbm.at[pl.ds(off, rows_per_chunk)])

    return gather


def kernel(z_e, embedding):
    d = embedding.shape[1]
    n = embedding.shape[0]
    flat = z_e.reshape(-1, d)
    m = flat.shape[0]
    f2 = jnp.sum(flat ** 2, axis=1, keepdims=True)
    e2 = jnp.sum(embedding ** 2, axis=1).reshape(1, n)

    idx2, loss_sum = _dist_argmin(flat, embedding, f2, e2, bm=512)
    idx = idx2.reshape(m)

    z_q = _make_sc_gather(n, d, m)(embedding, idx)
    vq_loss = loss_sum[0, 0] * (1.25 / (m * d))
    return z_q.reshape(z_e.shape), idx, vq_loss


# speedup vs baseline: 1.1077x; 1.1077x over previous
import functools

import jax
import jax.numpy as jnp
from jax import lax
from jax.experimental import pallas as pl
from jax.experimental.pallas import tpu as pltpu
from jax.experimental.pallas import tpu_sc as plsc


def _dist_argmin_body(f_ref, e_ref, f2_ref, e2_ref, idx_ref, acc_ref):
    n = e_ref.shape[0]
    dot = lax.dot_general(2.0 * f_ref[...], e_ref[...],
                          (((1,), (1,)), ((), ())),
                          preferred_element_type=jnp.float32)
    dist = (f2_ref[...] - dot) + e2_ref[...]
    m = jnp.min(dist, axis=1, keepdims=True)
    iota = lax.broadcasted_iota(jnp.int32, dist.shape, 1)
    idx_ref[...] = jnp.min(jnp.where(dist == m, iota, n), axis=1, keepdims=True)

    @pl.when(pl.program_id(0) == 0)
    def _():
        acc_ref[0, 0] = 0.0

    acc_ref[0, 0] += jnp.sum(m)


def _dist_argmin(flat, embedding, f2, e2, bm):
    m, d = flat.shape
    n = embedding.shape[0]
    return pl.pallas_call(
        _dist_argmin_body,
        grid=(m // bm,),
        in_specs=[
            pl.BlockSpec((bm, d), lambda i: (i, 0)),
            pl.BlockSpec((n, d), lambda i: (0, 0)),
            pl.BlockSpec((bm, 1), lambda i: (i, 0)),
            pl.BlockSpec((1, n), lambda i: (0, 0)),
        ],
        out_specs=[
            pl.BlockSpec((bm, 1), lambda i: (i, 0)),
            pl.BlockSpec((1, 1), lambda i: (0, 0), memory_space=pltpu.SMEM),
        ],
        out_shape=[
            jax.ShapeDtypeStruct((m, 1), jnp.int32),
            jax.ShapeDtypeStruct((1, 1), jnp.float32),
        ],
    )(flat, embedding, f2, e2)


def _make_sc_gather(v, d, b):
    info = plsc.get_sparse_core_info()
    nc, ns = info.num_cores, info.num_subcores
    nw = nc * ns
    assert b % (8 * nw) == 0
    b_per_w = b // nw
    chunks = 1
    while (b_per_w // chunks) * d * 4 > 300_000 or b_per_w % chunks:
        chunks += 1
    rows_per_chunk = b_per_w // chunks
    mesh = plsc.VectorSubcoreMesh(core_axis_name="c", subcore_axis_name="s")

    @functools.partial(
        pl.kernel,
        out_type=jax.ShapeDtypeStruct((b, d), jnp.float32),
        mesh=mesh,
        scratch_types=[
            pltpu.VMEM((b_per_w,), jnp.int32),
            pltpu.VMEM((rows_per_chunk, d), jnp.float32),
            pltpu.SemaphoreType.DMA,
        ],
    )
    def gather(table_hbm, idx_hbm, out_hbm, idx_v, rows_v, sem):
        wid = lax.axis_index("s") * nc + lax.axis_index("c")
        base = wid * b_per_w
        pltpu.sync_copy(idx_hbm.at[pl.ds(base, b_per_w)], idx_v)
        for c in range(chunks):
            sl = pl.ds(c * rows_per_chunk, rows_per_chunk)
            pltpu.async_copy(table_hbm.at[idx_v.at[sl]], rows_v, sem).wait()
            pltpu.sync_copy(rows_v, out_hbm.at[pl.ds(base + c * rows_per_chunk,
                                                     rows_per_chunk)])

    return gather


def kernel(z_e, embedding):
    d = embedding.shape[1]
    n = embedding.shape[0]
    flat = z_e.reshape(-1, d)
    m = flat.shape[0]
    f2 = jnp.sum(flat ** 2, axis=1, keepdims=True)
    e2 = jnp.sum(embedding ** 2, axis=1).reshape(1, n)

    idx2, loss_sum = _dist_argmin(flat, embedding, f2, e2, bm=512)
    idx = idx2.reshape(m)

    z_q = _make_sc_gather(n, d, m)(embedding, idx)
    vq_loss = loss_sum[0, 0] * (1.25 / (m * d))
    return z_q.reshape(z_e.shape), idx, vq_loss


# speedup vs baseline: 1.1472x; 1.0357x over previous
import functools

import jax
import jax.numpy as jnp
from jax import lax
from jax.experimental import pallas as pl
from jax.experimental.pallas import tpu as pltpu
from jax.experimental.pallas import tpu_sc as plsc


def _dist_argmin_body(f_ref, e_ref, f2_ref, e2_ref, idx_ref, acc_ref):
    n = e_ref.shape[0]
    dot = lax.dot_general(2.0 * f_ref[...], e_ref[...],
                          (((1,), (1,)), ((), ())),
                          preferred_element_type=jnp.float32)
    dist = (f2_ref[...] - dot) + e2_ref[...]
    m = jnp.min(dist, axis=1, keepdims=True)
    iota = lax.broadcasted_iota(jnp.int32, dist.shape, 1)
    idx_ref[...] = jnp.min(jnp.where(dist == m, iota, n), axis=1, keepdims=True)

    @pl.when(pl.program_id(0) == 0)
    def _():
        acc_ref[0, 0] = 0.0

    acc_ref[0, 0] += jnp.sum(m)


def _dist_argmin(flat, embedding, f2, e2, bm):
    m, d = flat.shape
    n = embedding.shape[0]
    return pl.pallas_call(
        _dist_argmin_body,
        grid=(m // bm,),
        in_specs=[
            pl.BlockSpec((bm, d), lambda i: (i, 0)),
            pl.BlockSpec((n, d), lambda i: (0, 0)),
            pl.BlockSpec((bm, 1), lambda i: (i, 0)),
            pl.BlockSpec((1, n), lambda i: (0, 0)),
        ],
        out_specs=[
            pl.BlockSpec((bm, 1), lambda i: (i, 0)),
            pl.BlockSpec((1, 1), lambda i: (0, 0), memory_space=pltpu.SMEM),
        ],
        out_shape=[
            jax.ShapeDtypeStruct((m, 1), jnp.int32),
            jax.ShapeDtypeStruct((1, 1), jnp.float32),
        ],
    )(flat, embedding, f2, e2)


def _make_sc_gather(v, d, b):
    info = plsc.get_sparse_core_info()
    nc, ns = info.num_cores, info.num_subcores
    nw = nc * ns
    assert b % (8 * nw) == 0
    b_per_w = b // nw
    chunks = 1
    while (b_per_w // chunks) * d * 4 > 300_000 or b_per_w % chunks:
        chunks += 1
    rows_per_chunk = b_per_w // chunks
    mesh = plsc.VectorSubcoreMesh(core_axis_name="c", subcore_axis_name="s")

    @functools.partial(
        pl.kernel,
        out_type=jax.ShapeDtypeStruct((b, d), jnp.float32),
        mesh=mesh,
        scratch_types=[
            pltpu.VMEM((b_per_w,), jnp.int32),
            pltpu.VMEM((rows_per_chunk, d), jnp.float32),
            pltpu.SemaphoreType.DMA,
        ],
    )
    def gather(table_hbm, idx_hbm, out_hbm, idx_v, rows_v, sem):
        wid = lax.axis_index("s") * nc + lax.axis_index("c")
        base = wid * b_per_w
        pltpu.sync_copy(idx_hbm.at[pl.ds(base, b_per_w)], idx_v)
        for c in range(chunks):
            sl = pl.ds(c * rows_per_chunk, rows_per_chunk)
            pltpu.async_copy(table_hbm.at[idx_v.at[sl]], rows_v, sem).wait()
            pltpu.sync_copy(rows_v, out_hbm.at[pl.ds(base + c * rows_per_chunk,
                                                     rows_per_chunk)])

    return gather


def kernel(z_e, embedding):
    d = embedding.shape[1]
    n = embedding.shape[0]
    flat = z_e.reshape(-1, d)
    m = flat.shape[0]
    f2 = jnp.sum(flat ** 2, axis=1, keepdims=True)
    e2 = jnp.sum(embedding ** 2, axis=1).reshape(1, n)

    idx2, loss_sum = _dist_argmin(flat, embedding, f2, e2, bm=1024)
    idx = idx2.reshape(m)

    z_q = _make_sc_gather(n, d, m)(embedding, idx)
    vq_loss = loss_sum[0, 0] * (1.25 / (m * d))
    return z_q.reshape(z_e.shape), idx, vq_loss
